# Initial kernel scaffold; baseline (speedup 1.0000x reference)
import jax
import jax.numpy as jnp
from jax.experimental import pallas as pl


def kernel(x, A_hat, W1, b1, W2, b2, gamma):
    raise NotImplementedError("write your pallas kernel here")


# speedup vs baseline: 1.3237x; 1.3237x over previous
import jax
import jax.numpy as jnp
from jax.experimental import pallas as pl
from jax.experimental.pallas import tpu as pltpu


def _pick_tile(n, align, cap):
    for r in range(min(cap, n), 0, -1):
        if r % align == 0 and n % r == 0:
            return r
    return n


def _encoder_body(x_ref, w1_ref, b1_ref, w2_ref, b2_ref, h0_ref, h0b_ref):
    h = jnp.maximum(
        jnp.dot(x_ref[...], w1_ref[...], preferred_element_type=jnp.float32)
        + b1_ref[...], 0.0)
    h0 = jnp.dot(h, w2_ref[...], preferred_element_type=jnp.float32) + b2_ref[...]
    h0_ref[...] = h0
    h0b_ref[...] = h0.astype(jnp.bfloat16)


def _hop1_body(gamma_ref, a_ref, h0b_ref, h0f_ref, abf_ref, h1b_ref, zp_ref):
    a16 = a_ref[...].astype(jnp.bfloat16)
    abf_ref[...] = a16
    part = jnp.dot(a16, h0b_ref[...], preferred_element_type=jnp.float32)
    h1b_ref[...] = part.astype(jnp.bfloat16)
    zp_ref[...] = gamma_ref[0] * h0f_ref[...] + gamma_ref[1] * part


def _prop_body(gamma_ref, abf_ref, h1b_ref, zp_ref, z_ref, hs0, hs1, *, R):
    k = pl.program_id(0)
    i = pl.program_id(1)

    @pl.when((k == 0) & (i == 0))
    def _():
        hs0[...] = h1b_ref[...]

    a = abf_ref[...]
    part = jax.lax.cond(
        k % 2 == 0,
        lambda: jnp.dot(a, hs0[...], preferred_element_type=jnp.float32),
        lambda: jnp.dot(a, hs1[...], preferred_element_type=jnp.float32))

    rows = pl.ds(i * R, R)

    @pl.when(k % 2 == 0)
    def _():
        hs1[rows, :] = part.astype(jnp.bfloat16)

    @pl.when(k % 2 == 1)
    def _():
        hs0[rows, :] = part.astype(jnp.bfloat16)

    g = gamma_ref[k + 2]

    @pl.when(k == 0)
    def _():
        z_ref[rows, :] = zp_ref[rows, :] + g * part

    @pl.when(k != 0)
    def _():
        z_ref[rows, :] = z_ref[rows, :] + g * part


def kernel(x, A_hat, W1, b1, W2, b2, gamma):
    N, IN_DIM = x.shape
    HID = W1.shape[1]
    C = W2.shape[1]
    KH = gamma.shape[0] - 1

    b1r = b1.reshape(1, HID)
    b2r = b2.reshape(1, C)

    R1 = _pick_tile(N, 8, 2000)
    h0f, h0b = pl.pallas_call(
        _encoder_body,
        grid=(N // R1,),
        in_specs=[
            pl.BlockSpec((R1, IN_DIM), lambda i: (i, 0)),
            pl.BlockSpec((IN_DIM, HID), lambda i: (0, 0)),
            pl.BlockSpec((1, HID), lambda i: (0, 0)),
            pl.BlockSpec((HID, C), lambda i: (0, 0)),
            pl.BlockSpec((1, C), lambda i: (0, 0)),
        ],
        out_specs=[
            pl.BlockSpec((R1, C), lambda i: (i, 0)),
            pl.BlockSpec((R1, C), lambda i: (i, 0)),
        ],
        out_shape=[
            jax.ShapeDtypeStruct((N, C), jnp.float32),
            jax.ShapeDtypeStruct((N, C), jnp.bfloat16),
        ],
    )(x, W1, b1r, W2, b2r)

    R2 = _pick_tile(N, 16, 400)
    abf, h1b, zp = pl.pallas_call(
        _hop1_body,
        grid=(N // R2,),
        in_specs=[
            pl.BlockSpec(memory_space=pltpu.SMEM),
            pl.BlockSpec((R2, N), lambda i: (i, 0)),
            pl.BlockSpec((N, C), lambda i: (0, 0)),
            pl.BlockSpec((R2, C), lambda i: (i, 0)),
        ],
        out_specs=[
            pl.BlockSpec((R2, N), lambda i: (i, 0)),
            pl.BlockSpec((R2, C), lambda i: (i, 0)),
            pl.BlockSpec((R2, C), lambda i: (i, 0)),
        ],
        out_shape=[
            jax.ShapeDtypeStruct((N, N), jnp.bfloat16),
            jax.ShapeDtypeStruct((N, C), jnp.bfloat16),
            jax.ShapeDtypeStruct((N, C), jnp.float32),
        ],
    )(gamma, A_hat, h0b, h0f)

    if KH == 1:
        return zp

    import functools
    body = functools.partial(_prop_body, R=R2)
    z = pl.pallas_call(
        body,
        grid=(KH - 1, N // R2),
        in_specs=[
            pl.BlockSpec(memory_space=pltpu.SMEM),
            pl.BlockSpec((R2, N), lambda k, i: (i, 0)),
            pl.BlockSpec((N, C), lambda k, i: (0, 0)),
            pl.BlockSpec((N, C), lambda k, i: (0, 0)),
        ],
        out_specs=pl.BlockSpec((N, C), lambda k, i: (0, 0)),
        out_shape=jax.ShapeDtypeStruct((N, C), jnp.float32),
        scratch_shapes=[
            pltpu.VMEM((N, C), jnp.bfloat16),
            pltpu.VMEM((N, C), jnp.bfloat16),
        ],
    )(gamma, abf, h1b, zp)
    return z
